# baseline (speedup 1.0000x reference)
import jax
import jax.numpy as jnp
from jax import lax
from jax.experimental import pallas as pl
from jax.experimental.pallas import tpu as pltpu
from jax.experimental.pallas import tpu_sc as plsc

D_MODEL = 1024
GATE_DIM = D_MODEL // 2
POS_DIM = D_MODEL // 4
PARAM_DIM = D_MODEL - GATE_DIM - POS_DIM
SEQ_LEN = 8192
N_GATE_TYPES = 20

NC = 2
NS = 16
L = 16
NW = NC * NS
ROWS_PER_W = SEQ_LEN // NW
CHUNK = 64
NCHUNK = ROWS_PER_W // CHUNK


def _body(gid_h, x1_h, x2_h, x3_h, gate_h, wb_h, pos_h, out_h,
          posbuf, gatebuf, wbuf, gidb, x1b, x2b, x3b):
    wid = lax.axis_index("s") * NC + lax.axis_index("c")
    base = wid * ROWS_PER_W
    pltpu.sync_copy(gate_h, gatebuf)
    pltpu.sync_copy(wb_h, wbuf)

    def chunk_body(c, carry):
        rbase = base + c * CHUNK
        pltpu.sync_copy(pos_h.at[pl.ds(rbase * D_MODEL, CHUNK * D_MODEL)],
                        posbuf.at[pl.ds(0, CHUNK * D_MODEL)])
        pltpu.sync_copy(gid_h.at[pl.ds(rbase, CHUNK)],
                        gidb.at[pl.ds(0, CHUNK)])
        pltpu.sync_copy(x1_h.at[pl.ds(rbase, CHUNK)], x1b.at[pl.ds(0, CHUNK)])
        pltpu.sync_copy(x2_h.at[pl.ds(rbase, CHUNK)], x2b.at[pl.ds(0, CHUNK)])
        pltpu.sync_copy(x3_h.at[pl.ds(rbase, CHUNK)], x3b.at[pl.ds(0, CHUNK)])

        def row_body(r, carry2):
            g = gidb[pl.ds(r, L)][0]
            v1 = jnp.full((L,), x1b[pl.ds(r, L)][0], jnp.float32)
            v2 = jnp.full((L,), x2b[pl.ds(r, L)][0], jnp.float32)
            v3 = jnp.full((L,), x3b[pl.ds(r, L)][0], jnp.float32)
            ro = r * D_MODEL
            go = g * GATE_DIM
            for k in range(GATE_DIM // L):
                sl = pl.ds(ro + k * L, L)
                posbuf[sl] = posbuf[sl] + gatebuf[pl.ds(go + k * L, L)]
            for k in range(POS_DIM // L):
                sl = pl.ds(ro + GATE_DIM + k * L, L)
                w0 = wbuf[pl.ds(k * L, L)]
                w1 = wbuf[pl.ds(POS_DIM + k * L, L)]
                bb = wbuf[pl.ds(2 * POS_DIM + k * L, L)]
                posbuf[sl] = posbuf[sl] + (v1 * w0 + v2 * w1 + bb)
            for k in range(PARAM_DIM // L):
                sl = pl.ds(ro + GATE_DIM + POS_DIM + k * L, L)
                wq = wbuf[pl.ds(3 * POS_DIM + k * L, L)]
                bq = wbuf[pl.ds(3 * POS_DIM + PARAM_DIM + k * L, L)]
                posbuf[sl] = posbuf[sl] + (v3 * wq + bq)
            return carry2

        lax.fori_loop(0, CHUNK, row_body, 0)
        pltpu.sync_copy(posbuf.at[pl.ds(0, CHUNK * D_MODEL)],
                        out_h.at[pl.ds(rbase * D_MODEL, CHUNK * D_MODEL)])
        return carry

    lax.fori_loop(0, NCHUNK, chunk_body, 0)


_sc_call = pl.kernel(
    _body,
    out_type=jax.ShapeDtypeStruct((SEQ_LEN * D_MODEL,), jnp.float32),
    mesh=plsc.VectorSubcoreMesh(core_axis_name="c", subcore_axis_name="s",
                                num_cores=NC, num_subcores=NS),
    scratch_types=[
        pltpu.VMEM((CHUNK * D_MODEL,), jnp.float32),
        pltpu.VMEM((N_GATE_TYPES * GATE_DIM,), jnp.float32),
        pltpu.VMEM((3 * POS_DIM + 2 * PARAM_DIM,), jnp.float32),
        pltpu.VMEM((CHUNK + L,), jnp.int32),
        pltpu.VMEM((CHUNK + L,), jnp.float32),
        pltpu.VMEM((CHUNK + L,), jnp.float32),
        pltpu.VMEM((CHUNK + L,), jnp.float32),
    ],
)


def kernel(x, gate_table, pos_table, W_pos, b_pos, W_param, b_param):
    xT = x.T
    gid = xT[0].astype(jnp.int32)
    wb = jnp.concatenate(
        [W_pos[:, 0], W_pos[:, 1], b_pos, W_param[:, 0], b_param])
    out = _sc_call(gid, xT[1], xT[2], xT[3], gate_table.reshape(-1), wb,
                   pos_table.reshape(-1))
    return out.reshape(SEQ_LEN, D_MODEL)


# speedup vs baseline: 2.7427x; 2.7427x over previous
import jax
import jax.numpy as jnp
from jax import lax
from jax.experimental import pallas as pl
from jax.experimental.pallas import tpu as pltpu
from jax.experimental.pallas import tpu_sc as plsc

D_MODEL = 1024
GATE_DIM = D_MODEL // 2
POS_DIM = D_MODEL // 4
PARAM_DIM = D_MODEL - GATE_DIM - POS_DIM
SEQ_LEN = 8192
N_GATE_TYPES = 20

NC = 2
NS = 16
L = 16
NW = NC * NS
ROWS_PER_W = SEQ_LEN // NW
CHUNK = 16
NCHUNKS = ROWS_PER_W // CHUNK
NBUF = 4


def _body(x_h, gid_h, gate_h, w_h, pos_h, out_h,
          b0, b1, b2, b3, gatebuf, wbuf, xbuf, gidb,
          si0, si1, si2, si3, so0, so1, so2, so3):
    wid = lax.axis_index("s") * NC + lax.axis_index("c")
    base = wid * ROWS_PER_W
    bufs = (b0, b1, b2, b3)
    sin = (si0, si1, si2, si3)
    sout = (so0, so1, so2, so3)

    pltpu.sync_copy(gate_h, gatebuf)
    pltpu.sync_copy(w_h, wbuf)
    pltpu.sync_copy(x_h.at[pl.ds(base * 4, ROWS_PER_W * 4)], xbuf)
    pltpu.sync_copy(gid_h.at[pl.ds(base, ROWS_PER_W)],
                    gidb.at[pl.ds(0, ROWS_PER_W)])

    W0 = [wbuf[pl.ds(k * L, L)] for k in range(POS_DIM // L)]
    W1 = [wbuf[pl.ds(POS_DIM + k * L, L)] for k in range(POS_DIM // L)]
    WQ = [wbuf[pl.ds(2 * POS_DIM + k * L, L)] for k in range(PARAM_DIM // L)]
    vc = [jnp.full((L,), c, jnp.int32) for c in range(4)]

    def in_slice(c):
        return pos_h.at[pl.ds(base + c * CHUNK, CHUNK), :]

    def out_slice(c):
        return out_h.at[pl.ds(base + c * CHUNK, CHUNK), :]

    USE_RING = True
    if USE_RING:
        for j in range(NBUF - 1):
            pltpu.async_copy(in_slice(j), bufs[j], sin[j])

    def compute(buf, c):
        @plsc.parallel_loop(0, CHUNK)
        def row(r):
            row = c * CHUNK + r
            vr = jnp.full((L,), row * 4, jnp.int32)
            v1 = plsc.load_gather(xbuf, [vr + vc[1]])
            v2 = plsc.load_gather(xbuf, [vr + vc[2]])
            v3 = plsc.load_gather(xbuf, [vr + vc[3]])
            g = gidb[pl.ds(row, L)][0]
            for k in range(GATE_DIM // L):
                sl = pl.ds(k * L, L)
                plsc.addupdate(buf.at[r, sl], gatebuf[g, sl])
            for k in range(POS_DIM // L):
                sl = pl.ds(GATE_DIM + k * L, L)
                t = gatebuf[g, sl] + (v1 * W0[k] + v2 * W1[k])
                plsc.addupdate(buf.at[r, sl], t)
            for k in range(PARAM_DIM // L):
                sl = pl.ds(GATE_DIM + POS_DIM + k * L, L)
                t = gatebuf[g, sl] + v3 * WQ[k]
                plsc.addupdate(buf.at[r, sl], t)

    if USE_RING:
        def outer(i, carry):
            for j in range(NBUF):
                c = i * NBUF + j
                pltpu.make_async_copy(in_slice(c), bufs[j], sin[j]).wait()
                compute(bufs[j], c)
                pltpu.async_copy(bufs[j], out_slice(c), sout[j])
                jn = (j + NBUF - 1) % NBUF

                @pl.when((c >= 1) & (c <= NCHUNKS - NBUF))
                def _wait_prev_out():
                    pltpu.make_async_copy(bufs[jn], out_slice(c - 1),
                                          sout[jn]).wait()

                @pl.when(c <= NCHUNKS - NBUF)
                def _issue_next_in():
                    pltpu.async_copy(in_slice(c + NBUF - 1), bufs[jn],
                                     sin[jn])

            return carry

        lax.fori_loop(0, NCHUNKS // NBUF, outer, 0)
        for j in range(NBUF):
            pltpu.make_async_copy(bufs[j], out_slice(NCHUNKS - NBUF + j),
                                  sout[j]).wait()
    else:
        def outer_sync(i, carry):
            pltpu.sync_copy(in_slice(i), b0)
            compute(b0, i)
            pltpu.sync_copy(b0, out_slice(i))
            return carry

        lax.fori_loop(0, NCHUNKS, outer_sync, 0)


_sc_call = pl.kernel(
    _body,
    out_type=jax.ShapeDtypeStruct((SEQ_LEN, D_MODEL), jnp.float32),
    mesh=plsc.VectorSubcoreMesh(core_axis_name="c", subcore_axis_name="s",
                                num_cores=NC, num_subcores=NS),
    compiler_params=pltpu.CompilerParams(needs_layout_passes=False),
    scratch_types=(
        [pltpu.VMEM((CHUNK, D_MODEL), jnp.float32) for _ in range(NBUF)]
        + [pltpu.VMEM((24, D_MODEL), jnp.float32),
           pltpu.VMEM((2 * POS_DIM + PARAM_DIM,), jnp.float32),
           pltpu.VMEM((ROWS_PER_W * 4,), jnp.float32),
           pltpu.VMEM((ROWS_PER_W + L,), jnp.int32)]
        + [pltpu.SemaphoreType.DMA for _ in range(2 * NBUF)]
    ),
)


def kernel(x, gate_table, pos_table, W_pos, b_pos, W_param, b_param):
    bias = jnp.concatenate([b_pos, b_param])
    gate_ext = jnp.concatenate(
        [gate_table, jnp.tile(bias[None, :], (N_GATE_TYPES, 1))], axis=1)
    gate_ext = jnp.pad(gate_ext, ((0, 24 - N_GATE_TYPES), (0, 0)))
    wcat = jnp.concatenate([W_pos[:, 0], W_pos[:, 1], W_param[:, 0]])
    gid = x[:, 0].astype(jnp.int32)
    return _sc_call(x.reshape(-1), gid, gate_ext, wcat, pos_table)


# speedup vs baseline: 3.0917x; 1.1272x over previous
import jax
import jax.numpy as jnp
from jax import lax
from jax.experimental import pallas as pl
from jax.experimental.pallas import tpu as pltpu
from jax.experimental.pallas import tpu_sc as plsc

D_MODEL = 1024
GATE_DIM = D_MODEL // 2
POS_DIM = D_MODEL // 4
PARAM_DIM = D_MODEL - GATE_DIM - POS_DIM
SEQ_LEN = 8192
N_GATE_TYPES = 20

NC = 2
NS = 16
L = 16
NW = NC * NS
ROWS_PER_W = SEQ_LEN // NW
CHUNK = 16
NCHUNKS = ROWS_PER_W // CHUNK
NBUF = 4


def _body(x_h, gid_h, gate_h, w_h, pos_h, out_h,
          b0, b1, b2, b3, gatebuf, wbuf, xbuf, gidb,
          si0, si1, si2, si3, so0, so1, so2, so3):
    wid = lax.axis_index("s") * NC + lax.axis_index("c")
    base = wid * ROWS_PER_W
    bufs = (b0, b1, b2, b3)
    sin = (si0, si1, si2, si3)
    sout = (so0, so1, so2, so3)

    pltpu.sync_copy(gate_h, gatebuf)
    pltpu.sync_copy(w_h, wbuf)
    pltpu.sync_copy(x_h.at[pl.ds(base * 4, ROWS_PER_W * 4)], xbuf)
    pltpu.sync_copy(gid_h.at[pl.ds(base, ROWS_PER_W)],
                    gidb.at[pl.ds(0, ROWS_PER_W)])

    vc = [jnp.full((L,), c, jnp.int32) for c in range(4)]

    def in_slice(c):
        return pos_h.at[pl.ds(base + c * CHUNK, CHUNK), :]

    def out_slice(c):
        return out_h.at[pl.ds(base + c * CHUNK, CHUNK), :]

    USE_RING = True
    if USE_RING:
        for j in range(NBUF - 1):
            pltpu.async_copy(in_slice(j), bufs[j], sin[j])

    def compute(buf, c):
        @plsc.parallel_loop(0, CHUNK)
        def row(r):
            ri = c * CHUNK + r
            vr = jnp.full((L,), ri * 4, jnp.int32)
            v1 = plsc.load_gather(xbuf, [vr + vc[1]])
            v2 = plsc.load_gather(xbuf, [vr + vc[2]])
            v3 = plsc.load_gather(xbuf, [vr + vc[3]])
            g = gidb[pl.ds(ri, L)][0]

            @plsc.parallel_loop(0, GATE_DIM // L, unroll=4)
            def gate_k(k):
                sl = pl.ds(k * L, L)
                plsc.addupdate(buf.at[r, sl], gatebuf[g, sl])

            @plsc.parallel_loop(0, POS_DIM // L, unroll=4)
            def pos_k(k):
                sl = pl.ds(GATE_DIM + k * L, L)
                t = gatebuf[g, sl] + (v1 * wbuf[pl.ds(k * L, L)]
                                      + v2 * wbuf[pl.ds(POS_DIM + k * L, L)])
                plsc.addupdate(buf.at[r, sl], t)

            @plsc.parallel_loop(0, PARAM_DIM // L, unroll=4)
            def param_k(k):
                sl = pl.ds(GATE_DIM + POS_DIM + k * L, L)
                t = gatebuf[g, sl] + v3 * wbuf[pl.ds(2 * POS_DIM + k * L, L)]
                plsc.addupdate(buf.at[r, sl], t)

    if USE_RING:
        def outer(i, carry):
            for j in range(NBUF):
                c = i * NBUF + j
                pltpu.make_async_copy(in_slice(c), bufs[j], sin[j]).wait()
                compute(bufs[j], c)
                pltpu.async_copy(bufs[j], out_slice(c), sout[j])
                jn = (j + NBUF - 1) % NBUF

                @pl.when((c >= 1) & (c <= NCHUNKS - NBUF))
                def _wait_prev_out():
                    pltpu.make_async_copy(bufs[jn], out_slice(c - 1),
                                          sout[jn]).wait()

                @pl.when(c <= NCHUNKS - NBUF)
                def _issue_next_in():
                    pltpu.async_copy(in_slice(c + NBUF - 1), bufs[jn],
                                     sin[jn])

            return carry

        lax.fori_loop(0, NCHUNKS // NBUF, outer, 0)
        for j in range(NBUF):
            pltpu.make_async_copy(bufs[j], out_slice(NCHUNKS - NBUF + j),
                                  sout[j]).wait()
    else:
        def outer_sync(i, carry):
            pltpu.sync_copy(in_slice(i), b0)
            compute(b0, i)
            pltpu.sync_copy(b0, out_slice(i))
            return carry

        lax.fori_loop(0, NCHUNKS, outer_sync, 0)


_sc_call = pl.kernel(
    _body,
    out_type=jax.ShapeDtypeStruct((SEQ_LEN, D_MODEL), jnp.float32),
    mesh=plsc.VectorSubcoreMesh(core_axis_name="c", subcore_axis_name="s",
                                num_cores=NC, num_subcores=NS),
    compiler_params=pltpu.CompilerParams(needs_layout_passes=False),
    scratch_types=(
        [pltpu.VMEM((CHUNK, D_MODEL), jnp.float32) for _ in range(NBUF)]
        + [pltpu.VMEM((24, D_MODEL), jnp.float32),
           pltpu.VMEM((2 * POS_DIM + PARAM_DIM,), jnp.float32),
           pltpu.VMEM((ROWS_PER_W * 4,), jnp.float32),
           pltpu.VMEM((ROWS_PER_W + L,), jnp.int32)]
        + [pltpu.SemaphoreType.DMA for _ in range(2 * NBUF)]
    ),
)


def kernel(x, gate_table, pos_table, W_pos, b_pos, W_param, b_param):
    bias = jnp.concatenate([b_pos, b_param])
    gate_ext = jnp.concatenate(
        [gate_table, jnp.tile(bias[None, :], (N_GATE_TYPES, 1))], axis=1)
    gate_ext = jnp.pad(gate_ext, ((0, 24 - N_GATE_TYPES), (0, 0)))
    wcat = jnp.concatenate([W_pos[:, 0], W_pos[:, 1], W_param[:, 0]])
    gid = x[:, 0].astype(jnp.int32)
    return _sc_call(x.reshape(-1), gid, gate_ext, wcat, pos_table)


# speedup vs baseline: 3.1535x; 1.0200x over previous
import jax
import jax.numpy as jnp
from jax import lax
from jax.experimental import pallas as pl
from jax.experimental.pallas import tpu as pltpu
from jax.experimental.pallas import tpu_sc as plsc

D_MODEL = 1024
GATE_DIM = D_MODEL // 2
POS_DIM = D_MODEL // 4
PARAM_DIM = D_MODEL - GATE_DIM - POS_DIM
SEQ_LEN = 8192
N_GATE_TYPES = 20

NC = 2
NS = 16
L = 16
NW = NC * NS
ROWS_PER_W = SEQ_LEN // NW
CHUNK = 16
NCHUNKS = ROWS_PER_W // CHUNK
NBUF = 4


def _body(x_h, gate_h, w_h, bp_h, bq_h, pos_h, out_h,
          b0, b1, b2, b3, gatebuf, wbuf, biasbuf, xbuf, gidb,
          si0, si1, si2, si3, so0, so1, so2, so3):
    wid = lax.axis_index("s") * NC + lax.axis_index("c")
    base = wid * ROWS_PER_W
    bufs = (b0, b1, b2, b3)
    sin = (si0, si1, si2, si3)
    sout = (so0, so1, so2, so3)

    pltpu.sync_copy(gate_h, gatebuf)
    pltpu.sync_copy(w_h, wbuf)
    pltpu.sync_copy(bp_h, biasbuf.at[pl.ds(0, POS_DIM)])
    pltpu.sync_copy(bq_h, biasbuf.at[pl.ds(POS_DIM, PARAM_DIM)])
    pltpu.sync_copy(x_h.at[pl.ds(base * 4, ROWS_PER_W * 4)], xbuf)
    pltpu.sync_copy(x_h.at[pl.ds(SEQ_LEN * 4 + base, ROWS_PER_W)],
                    gidb.at[pl.ds(0, ROWS_PER_W)])

    vc = [jnp.full((L,), c, jnp.int32) for c in range(4)]

    def in_slice(c):
        return pos_h.at[pl.ds(base + c * CHUNK, CHUNK), :]

    def out_slice(c):
        return out_h.at[pl.ds(base + c * CHUNK, CHUNK), :]

    USE_RING = True
    if USE_RING:
        for j in range(NBUF - 1):
            pltpu.async_copy(in_slice(j), bufs[j], sin[j])

    def compute(buf, c):
        @plsc.parallel_loop(0, CHUNK)
        def row(r):
            ri = c * CHUNK + r
            vr = jnp.full((L,), ri * 4, jnp.int32)
            v1 = plsc.load_gather(xbuf, [vr + vc[1]])
            v2 = plsc.load_gather(xbuf, [vr + vc[2]])
            v3 = plsc.load_gather(xbuf, [vr + vc[3]])
            g = plsc.bitcast(gidb[pl.ds(ri, L)], jnp.int32)[0]

            @plsc.parallel_loop(0, GATE_DIM // L, unroll=8)
            def gate_k(k):
                sl = pl.ds(k * L, L)
                plsc.addupdate(buf.at[r, sl], gatebuf[g, sl])

            @plsc.parallel_loop(0, POS_DIM // L, unroll=8)
            def pos_k(k):
                sl = pl.ds(GATE_DIM + k * L, L)
                t = biasbuf[pl.ds(k * L, L)] + (
                    v1 * wbuf[pl.ds(k * L, L)]
                    + v2 * wbuf[pl.ds(POS_DIM + k * L, L)])
                plsc.addupdate(buf.at[r, sl], t)

            @plsc.parallel_loop(0, PARAM_DIM // L, unroll=8)
            def param_k(k):
                sl = pl.ds(GATE_DIM + POS_DIM + k * L, L)
                t = biasbuf[pl.ds(POS_DIM + k * L, L)] + (
                    v3 * wbuf[pl.ds(2 * POS_DIM + k * L, L)])
                plsc.addupdate(buf.at[r, sl], t)

    if USE_RING:
        def outer(i, carry):
            for j in range(NBUF):
                c = i * NBUF + j
                pltpu.make_async_copy(in_slice(c), bufs[j], sin[j]).wait()
                compute(bufs[j], c)
                pltpu.async_copy(bufs[j], out_slice(c), sout[j])
                jn = (j + NBUF - 1) % NBUF

                @pl.when((c >= 1) & (c <= NCHUNKS - NBUF))
                def _wait_prev_out():
                    pltpu.make_async_copy(bufs[jn], out_slice(c - 1),
                                          sout[jn]).wait()

                @pl.when(c <= NCHUNKS - NBUF)
                def _issue_next_in():
                    pltpu.async_copy(in_slice(c + NBUF - 1), bufs[jn],
                                     sin[jn])

            return carry

        lax.fori_loop(0, NCHUNKS // NBUF, outer, 0)
        for j in range(NBUF):
            pltpu.make_async_copy(bufs[j], out_slice(NCHUNKS - NBUF + j),
                                  sout[j]).wait()
    else:
        def outer_sync(i, carry):
            pltpu.sync_copy(in_slice(i), b0)
            compute(b0, i)
            pltpu.sync_copy(b0, out_slice(i))
            return carry

        lax.fori_loop(0, NCHUNKS, outer_sync, 0)


_sc_call = pl.kernel(
    _body,
    out_type=jax.ShapeDtypeStruct((SEQ_LEN, D_MODEL), jnp.float32),
    mesh=plsc.VectorSubcoreMesh(core_axis_name="c", subcore_axis_name="s",
                                num_cores=NC, num_subcores=NS),
    compiler_params=pltpu.CompilerParams(needs_layout_passes=False),
    scratch_types=(
        [pltpu.VMEM((CHUNK, D_MODEL), jnp.float32) for _ in range(NBUF)]
        + [pltpu.VMEM((24, GATE_DIM), jnp.float32),
           pltpu.VMEM((2 * POS_DIM + PARAM_DIM,), jnp.float32),
           pltpu.VMEM((POS_DIM + PARAM_DIM,), jnp.float32),
           pltpu.VMEM((ROWS_PER_W * 4,), jnp.float32),
           pltpu.VMEM((ROWS_PER_W + L,), jnp.float32)]
        + [pltpu.SemaphoreType.DMA for _ in range(2 * NBUF)]
    ),
)


def kernel(x, gate_table, pos_table, W_pos, b_pos, W_param, b_param):
    wcat = jnp.concatenate([W_pos[:, 0], W_pos[:, 1], W_param[:, 0]])
    gate_pad = jnp.pad(gate_table, ((0, 24 - N_GATE_TYPES), (0, 0)))
    gid_bits = jax.lax.bitcast_convert_type(
        x[:, 0].astype(jnp.int32), jnp.float32)
    xpack = jnp.concatenate([x.reshape(-1), gid_bits])
    return _sc_call(xpack, gate_pad, wcat, b_pos, b_param, pos_table)
